# Initial kernel scaffold; baseline (speedup 1.0000x reference)
import jax
import jax.numpy as jnp
from jax.experimental import pallas as pl


def kernel(x, emb0, emb1, emb2, emb3, emb4, emb5, emb6, emb7, emb8, W, b, gamma, beta):
    raise NotImplementedError("write your pallas kernel here")


# speedup vs baseline: 8.5103x; 8.5103x over previous
import functools

import jax
import jax.numpy as jnp
from jax.experimental import pallas as pl

_EMBED_DIMS = [(33, 17), (2, 1), (8, 4), (3, 2), (3, 2), (3, 2), (3, 2), (3, 2), (20, 10)]
_VOFFS = []
_COFFS = []
_v = 0
_c = 0
for _vv, _dd in _EMBED_DIMS:
    _VOFFS.append(_v)
    _COFFS.append(_c)
    _v += _vv
    _c += _dd
_TOTV = _v
_TOTC = _c
_KPAD = 128
_CPAD = 48


def _body(nb, xb_ref, xr_ref, e_ref, w_ref, b_ref, g_ref, be_ref, o_ref):
    xr = xr_ref[...]
    mean = jnp.sum(xr) * (1.0 / nb)
    var = jnp.sum((xr - mean) ** 2) * (1.0 / nb)
    rstd = jax.lax.rsqrt(var + 1e-5)

    xb = xb_ref[...]
    cn = (xb[:, 0:1] - mean) * rstd * g_ref[0, 0] + be_ref[0, 0]

    bb = xb.shape[0]
    iota = jax.lax.broadcasted_iota(jnp.int32, (bb, _KPAD), 1)
    g = jnp.zeros((bb, _KPAD), jnp.float32)
    for i, voff in enumerate(_VOFFS):
        xi = xb[:, i + 1:i + 2].astype(jnp.int32)
        g += (iota == xi + voff).astype(jnp.float32)
    g = jnp.where(iota == _TOTV, cn, g)

    m = jax.lax.dot_general(
        e_ref[...], w_ref[...], (((1,), (1,)), ((), ())),
        preferred_element_type=jnp.float32, precision=jax.lax.Precision.HIGHEST)
    z = jax.lax.dot_general(
        g, m, (((1,), (0,)), ((), ())),
        preferred_element_type=jnp.float32, precision=jax.lax.Precision.HIGHEST)
    z = z + b_ref[...]
    z = z - jnp.max(z, axis=1, keepdims=True)
    ez = jnp.exp(z)
    o_ref[...] = ez / jnp.sum(ez, axis=1, keepdims=True)


def kernel(x, emb0, emb1, emb2, emb3, emb4, emb5, emb6, emb7, emb8, W, b, gamma, beta):
    tables = [emb0, emb1, emb2, emb3, emb4, emb5, emb6, emb7, emb8]
    B = x.shape[0]
    d_out = W.shape[0]

    e = jnp.zeros((_KPAD, _CPAD), jnp.float32)
    for i, t in enumerate(tables):
        v, d = t.shape
        e = e.at[_VOFFS[i]:_VOFFS[i] + v, _COFFS[i]:_COFFS[i] + d].set(t)
    e = e.at[_TOTV, _TOTC].set(1.0)
    w_pad = jnp.zeros((d_out, _CPAD), jnp.float32).at[:, :_TOTC + 1].set(W)

    xr = x[:, 0].reshape(128, B // 128)
    b2 = b.reshape(1, d_out)
    g2 = gamma.reshape(1, 1)
    be2 = beta.reshape(1, 1)

    bb = 1024
    grid = (B // bb,)
    out = pl.pallas_call(
        functools.partial(_body, float(B)),
        grid=grid,
        in_specs=[
            pl.BlockSpec((bb, x.shape[1]), lambda i: (i, 0)),
            pl.BlockSpec(xr.shape, lambda i: (0, 0)),
            pl.BlockSpec(e.shape, lambda i: (0, 0)),
            pl.BlockSpec(w_pad.shape, lambda i: (0, 0)),
            pl.BlockSpec(b2.shape, lambda i: (0, 0)),
            pl.BlockSpec(g2.shape, lambda i: (0, 0)),
            pl.BlockSpec(be2.shape, lambda i: (0, 0)),
        ],
        out_specs=pl.BlockSpec((bb, d_out), lambda i: (i, 0)),
        out_shape=jax.ShapeDtypeStruct((B, d_out), jnp.float32),
    )(x, xr, e, w_pad, b2, g2, be2)
    return out
